# baseline (speedup 1.0000x reference)
import functools

import jax
import jax.numpy as jnp
from jax.experimental import pallas as pl

N_EXPERTS = 64
TOPK = 8
BLOCK_ROWS = 512
CHUNK_ROWS = 128


def _topk_softmax_t(logits_t):
    iota = jax.lax.broadcasted_iota(jnp.int32, logits_t.shape, 0)
    l = logits_t
    vals = []
    idxs = []
    for _ in range(TOPK):
        m = jnp.max(l, axis=0, keepdims=True)
        idx = jnp.min(
            jnp.where(l == m, iota, N_EXPERTS), axis=0, keepdims=True
        )
        vals.append(m)
        idxs.append(idx)
        l = jnp.where(iota == idx, -jnp.inf, l)

    v = jnp.concatenate(vals, axis=0)
    e = jnp.exp(v - vals[0])
    w = e / jnp.sum(e, axis=0, keepdims=True)
    return w, jnp.concatenate(idxs, axis=0)


def _router_kernel(x_ref, w_ref, wout_ref, iout_ref):
    for c in range(BLOCK_ROWS // CHUNK_ROWS):
        rows = pl.ds(c * CHUNK_ROWS, CHUNK_ROWS)
        logits_t = jax.lax.dot_general(
            w_ref[...],
            x_ref[rows, :],
            dimension_numbers=(((1,), (1,)), ((), ())),
            preferred_element_type=jnp.float32,
        )
        w, i = _topk_softmax_t(logits_t)
        wout_ref[rows, :] = w.T
        iout_ref[rows, :] = i.T


@jax.jit
def kernel(x, gate_w):
    n_rows = x.shape[0]
    grid = (n_rows // BLOCK_ROWS,)
    wout, iout = pl.pallas_call(
        _router_kernel,
        grid=grid,
        in_specs=[
            pl.BlockSpec((BLOCK_ROWS, x.shape[1]), lambda i: (i, 0)),
            pl.BlockSpec((N_EXPERTS, x.shape[1]), lambda i: (0, 0)),
        ],
        out_specs=[
            pl.BlockSpec((BLOCK_ROWS, TOPK), lambda i: (i, 0)),
            pl.BlockSpec((BLOCK_ROWS, TOPK), lambda i: (i, 0)),
        ],
        out_shape=[
            jax.ShapeDtypeStruct((n_rows, TOPK), jnp.float32),
            jax.ShapeDtypeStruct((n_rows, TOPK), jnp.int32),
        ],
    )(x, gate_w)
    return (wout, iout)


# speedup vs baseline: 1.0634x; 1.0634x over previous
import functools

import jax
import jax.numpy as jnp
from jax.experimental import pallas as pl

N_EXPERTS = 64
TOPK = 8
BLOCK_ROWS = 1024
CHUNK_ROWS = 128


def _topk_softmax_t(logits_t):
    iota = jax.lax.broadcasted_iota(jnp.int32, logits_t.shape, 0)
    l = logits_t
    vals = []
    idxs = []
    for _ in range(TOPK):
        m = jnp.max(l, axis=0, keepdims=True)
        idx = jnp.min(
            jnp.where(l == m, iota, N_EXPERTS), axis=0, keepdims=True
        )
        vals.append(m)
        idxs.append(idx)
        l = jnp.where(iota == idx, -jnp.inf, l)

    v = jnp.concatenate(vals, axis=0)
    e = jnp.exp(v - vals[0])
    w = e / jnp.sum(e, axis=0, keepdims=True)
    return w, jnp.concatenate(idxs, axis=0)


def _router_kernel(xa_ref, xb_ref, w_ref, wout_ref, iout_ref):
    half = BLOCK_ROWS // 2
    for h, x_ref in ((0, xa_ref), (1, xb_ref)):
        for c in range(half // CHUNK_ROWS):
            rows = pl.ds(c * CHUNK_ROWS, CHUNK_ROWS)
            out_rows = pl.ds(h * half + c * CHUNK_ROWS, CHUNK_ROWS)
            logits_t = jax.lax.dot_general(
                w_ref[...],
                x_ref[rows, :],
                dimension_numbers=(((1,), (1,)), ((), ())),
                preferred_element_type=jnp.float32,
            )
            w, i = _topk_softmax_t(logits_t)
            wout_ref[out_rows, :] = w.T
            iout_ref[out_rows, :] = i.T


@jax.jit
def kernel(x, gate_w):
    n_rows = x.shape[0]
    grid = (n_rows // BLOCK_ROWS,)
    half = BLOCK_ROWS // 2
    wout, iout = pl.pallas_call(
        _router_kernel,
        grid=grid,
        in_specs=[
            pl.BlockSpec((half, x.shape[1]), lambda i: (2 * i, 0)),
            pl.BlockSpec((half, x.shape[1]), lambda i: (2 * i + 1, 0)),
            pl.BlockSpec((N_EXPERTS, x.shape[1]), lambda i: (0, 0)),
        ],
        out_specs=[
            pl.BlockSpec((BLOCK_ROWS, TOPK), lambda i: (i, 0)),
            pl.BlockSpec((BLOCK_ROWS, TOPK), lambda i: (i, 0)),
        ],
        out_shape=[
            jax.ShapeDtypeStruct((n_rows, TOPK), jnp.float32),
            jax.ShapeDtypeStruct((n_rows, TOPK), jnp.int32),
        ],
    )(x, x, gate_w)
    return (wout, iout)


# speedup vs baseline: 1.0651x; 1.0016x over previous
import functools

import jax
import jax.numpy as jnp
from jax.experimental import pallas as pl

N_EXPERTS = 64
TOPK = 8
BLOCK_ROWS = 1024
CHUNK_ROWS = 256


def _topk_softmax_t(logits_t):
    iota = jax.lax.broadcasted_iota(jnp.int32, logits_t.shape, 0)
    l = logits_t
    vals = []
    idxs = []
    for _ in range(TOPK):
        m = jnp.max(l, axis=0, keepdims=True)
        idx = jnp.min(
            jnp.where(l == m, iota, N_EXPERTS), axis=0, keepdims=True
        )
        vals.append(m)
        idxs.append(idx)
        l = jnp.where(iota == idx, -jnp.inf, l)

    v = jnp.concatenate(vals, axis=0)
    e = jnp.exp(v - vals[0])
    w = e / jnp.sum(e, axis=0, keepdims=True)
    return w, jnp.concatenate(idxs, axis=0)


def _router_kernel(x_ref, w_ref, wout_ref, iout_ref):
    for c in range(BLOCK_ROWS // CHUNK_ROWS):
        rows = pl.ds(c * CHUNK_ROWS, CHUNK_ROWS)
        logits_t = jax.lax.dot_general(
            w_ref[...],
            x_ref[rows, :],
            dimension_numbers=(((1,), (1,)), ((), ())),
            preferred_element_type=jnp.float32,
        )
        w, i = _topk_softmax_t(logits_t)
        wout_ref[rows, :] = w.T
        iout_ref[rows, :] = i.T


@jax.jit
def kernel(x, gate_w):
    n_rows = x.shape[0]
    grid = (n_rows // BLOCK_ROWS,)
    wout, iout = pl.pallas_call(
        _router_kernel,
        grid=grid,
        in_specs=[
            pl.BlockSpec((BLOCK_ROWS, x.shape[1]), lambda i: (i, 0)),
            pl.BlockSpec((N_EXPERTS, x.shape[1]), lambda i: (0, 0)),
        ],
        out_specs=[
            pl.BlockSpec((BLOCK_ROWS, TOPK), lambda i: (i, 0)),
            pl.BlockSpec((BLOCK_ROWS, TOPK), lambda i: (i, 0)),
        ],
        out_shape=[
            jax.ShapeDtypeStruct((n_rows, TOPK), jnp.float32),
            jax.ShapeDtypeStruct((n_rows, TOPK), jnp.int32),
        ],
    )(x, gate_w)
    return (wout, iout)
